# baseline (speedup 1.0000x reference)
import functools

import jax
import jax.numpy as jnp
from jax import lax
from jax.experimental import pallas as pl
from jax.experimental.pallas import tpu as pltpu
from jax.experimental.pallas import tpu_sc as plsc

_NC = 2
_NS = 16
_NW = _NC * _NS
_IDX_CHUNK = 128


@functools.lru_cache(maxsize=None)
def _build_gather(n):
    bpw = n // _NW
    n_chunks = bpw // _IDX_CHUNK
    mesh = plsc.VectorSubcoreMesh(core_axis_name="c", subcore_axis_name="s")

    @functools.partial(
        pl.kernel,
        mesh=mesh,
        compiler_params=pltpu.CompilerParams(use_tc_tiling_on_sc=True),
        out_type=(
            jax.ShapeDtypeStruct((n, 128), jnp.float32),
            jax.ShapeDtypeStruct((n, 128), jnp.float32),
            jax.ShapeDtypeStruct((n, 128), jnp.float32),
        ),
        scratch_types=[
            pltpu.VMEM((bpw,), jnp.int32),
            pltpu.VMEM((bpw,), jnp.int32),
            pltpu.VMEM((bpw,), jnp.int32),
            pltpu.VMEM((bpw, 128), jnp.float32),
            pltpu.SemaphoreType.DMA,
        ],
    )
    def gather3(item_idx, key_idx, genre_idx, item_emb2, key_emb2, genre_emb2,
                item_out, key_out, genre_out,
                idx_i, idx_k, idx_g, rows, sem):
        wid = lax.axis_index("s") * _NC + lax.axis_index("c")
        base = wid * bpw
        pltpu.sync_copy(item_idx.at[pl.ds(base, bpw)], idx_i)
        pltpu.sync_copy(key_idx.at[pl.ds(base, bpw)], idx_k)
        pltpu.sync_copy(genre_idx.at[pl.ds(base, bpw)], idx_g)
        for tab, idx, out in ((item_emb2, idx_i, item_out),
                              (key_emb2, idx_k, key_out),
                              (genre_emb2, idx_g, genre_out)):
            copies = []
            for j in range(n_chunks):
                sl = pl.ds(j * _IDX_CHUNK, _IDX_CHUNK)
                copies.append(pltpu.async_copy(tab.at[idx.at[sl]],
                                               rows.at[sl], sem))
            for c in copies:
                c.wait()
            pltpu.sync_copy(rows, out.at[pl.ds(base, bpw)])

    return gather3


def _gelu(x):
    return 0.5 * x * (1.0 + lax.erf(x * 0.7071067811865476))


def _dot(a, b):
    return jnp.dot(a, b, precision=lax.Precision.HIGHEST,
                   preferred_element_type=jnp.float32)


def _sel16(x2, sel):
    out = x2[:, 0:16]
    for j in range(1, 8):
        out = jnp.where(sel == j, x2[:, 16 * j:16 * (j + 1)], out)
    return out


def _mlp_body(items2, keys2, genres2, iid, kid, gid, audio, wa, ba,
              w1i, w1k, w1g, w1a, b1, w2, b2, out):
    items = jnp.where((iid[...] & 1) == 1, items2[:, 64:128], items2[:, 0:64])
    keys = _sel16(keys2[...], kid[...] & 7)
    genres = _sel16(genres2[...], gid[...] & 7)
    a = _gelu(_dot(audio[...], wa[...]) + ba[...])
    h = _dot(items, w1i[...])
    h = h + _dot(keys, w1k[...])
    h = h + _dot(genres, w1g[...])
    h = h + _dot(a, w1a[...])
    h = _gelu(h + b1[...])
    out[...] = _dot(h, w2[...]) + b2[...]


def kernel(item_ids, key_ids, genre_ids, audio_cont, item_emb, key_emb,
           genre_emb, W_audio, b_audio, W1, b1, W2, b2):
    n = item_ids.shape[0]
    d_item = item_emb.shape[1]
    d_small = key_emb.shape[1]
    d_audio = W_audio.shape[1]
    d_hid = W1.shape[1]
    d_out = W2.shape[1]

    item_emb2 = item_emb.reshape(-1, 128)
    key_emb2 = key_emb.reshape(-1, 128)
    genre_emb2 = genre_emb.reshape(-1, 128)
    item_ids = item_ids.astype(jnp.int32)
    key_ids = key_ids.astype(jnp.int32)
    genre_ids = genre_ids.astype(jnp.int32)

    gather3 = _build_gather(n)
    items2, keys2, genres2 = gather3(
        lax.shift_right_logical(item_ids, 1),
        lax.shift_right_logical(key_ids, 3),
        lax.shift_right_logical(genre_ids, 3),
        item_emb2, key_emb2, genre_emb2)

    w1i = W1[:d_item]
    w1k = W1[d_item:d_item + d_small]
    w1g = W1[d_item + d_small:d_item + 2 * d_small]
    w1a = W1[d_item + 2 * d_small:]

    bn = min(n, 2048)
    grid = (n // bn,)

    def row_spec(d):
        return pl.BlockSpec((bn, d), lambda i: (i, 0))

    def rep_spec(r, c):
        return pl.BlockSpec((r, c), lambda i: (0, 0))

    return pl.pallas_call(
        _mlp_body,
        grid=grid,
        in_specs=[
            row_spec(128), row_spec(128), row_spec(128),
            row_spec(1), row_spec(1), row_spec(1),
            row_spec(audio_cont.shape[1]),
            rep_spec(W_audio.shape[0], d_audio), rep_spec(1, d_audio),
            rep_spec(d_item, d_hid), rep_spec(d_small, d_hid),
            rep_spec(d_small, d_hid), rep_spec(d_audio, d_hid),
            rep_spec(1, d_hid),
            rep_spec(d_hid, d_out), rep_spec(1, d_out),
        ],
        out_specs=row_spec(d_out),
        out_shape=jax.ShapeDtypeStruct((n, d_out), jnp.float32),
    )(items2, keys2, genres2,
      item_ids.reshape(n, 1), key_ids.reshape(n, 1), genre_ids.reshape(n, 1),
      audio_cont,
      W_audio, b_audio.reshape(1, -1),
      w1i, w1k, w1g, w1a, b1.reshape(1, -1),
      W2, b2.reshape(1, -1))


# speedup vs baseline: 2.0276x; 2.0276x over previous
import functools

import jax
import jax.numpy as jnp
from jax import lax
from jax.experimental import pallas as pl
from jax.experimental.pallas import tpu as pltpu
from jax.experimental.pallas import tpu_sc as plsc

_NC = 2
_NS = 16
_NW = _NC * _NS


@functools.lru_cache(maxsize=None)
def _build_gather(n, d_item, d_small):
    bpw = n // _NW
    d_pack = 128
    mesh = plsc.VectorSubcoreMesh(core_axis_name="c", subcore_axis_name="s")

    @functools.partial(
        pl.kernel,
        mesh=mesh,
        compiler_params=pltpu.CompilerParams(use_tc_tiling_on_sc=True),
        out_type=(
            jax.ShapeDtypeStruct((n, d_item), jnp.float32),
            jax.ShapeDtypeStruct((n, d_small), jnp.float32),
            jax.ShapeDtypeStruct((n, d_small), jnp.float32),
        ),
        scratch_types=[
            pltpu.VMEM((bpw,), jnp.int32),
            pltpu.VMEM((bpw,), jnp.int32),
            pltpu.VMEM((bpw,), jnp.int32),
            pltpu.VMEM((bpw // 4, d_item), jnp.float32),
            pltpu.VMEM((bpw // 4, d_small), jnp.float32),
            pltpu.VMEM((bpw // 4, d_small), jnp.float32),
            pltpu.VMEM((bpw // 4, d_item), jnp.float32),
            pltpu.VMEM((bpw // 4, d_small), jnp.float32),
            pltpu.VMEM((bpw // 4, d_small), jnp.float32),
            pltpu.SemaphoreType.DMA,
            pltpu.SemaphoreType.DMA,
        ],
    )
    def gatherpack(item_ids, key_ids, genre_ids, item_emb, key_emb, genre_emb,
                   item_out, key_out, genre_out,
                   sid_i, sid_k, sid_g,
                   ri0, rk0, rg0, ri1, rk1, rg1, sem0, sem1):
        wid = lax.axis_index("s") * _NC + lax.axis_index("c")
        base = wid * bpw
        quarter = bpw // 4
        pltpu.sync_copy(item_ids.at[pl.ds(base, bpw)], sid_i)
        pltpu.sync_copy(key_ids.at[pl.ds(base, bpw)], sid_k)
        pltpu.sync_copy(genre_ids.at[pl.ds(base, bpw)], sid_g)

        bufsets = ((ri0, rk0, rg0, sem0), (ri1, rk1, rg1, sem1))

        def issue(p):
            rows_i, rows_k, rows_g, sem = bufsets[p % 2]
            lo = p * quarter

            def body(g, carry):
                iv = sid_i[pl.ds(lo + g * 16, 16)]
                kv = sid_k[pl.ds(lo + g * 16, 16)]
                gv = sid_g[pl.ds(lo + g * 16, 16)]
                for j in range(16):
                    r = g * 16 + j
                    pltpu.async_copy(item_emb.at[pl.ds(iv[j], 1), :],
                                     rows_i.at[pl.ds(r, 1), :], sem)
                    pltpu.async_copy(key_emb.at[pl.ds(kv[j], 1), :],
                                     rows_k.at[pl.ds(r, 1), :], sem)
                    pltpu.async_copy(genre_emb.at[pl.ds(gv[j], 1), :],
                                     rows_g.at[pl.ds(r, 1), :], sem)
                return carry

            lax.fori_loop(0, quarter // 16, body, 0)

        def flush(p):
            rows_i, rows_k, rows_g, sem = bufsets[p % 2]
            off = p * quarter
            pltpu.make_async_copy(item_out.at[pl.ds(0, quarter)],
                                  rows_i, sem).wait()
            pltpu.make_async_copy(key_out.at[pl.ds(0, quarter)],
                                  rows_k, sem).wait()
            pltpu.make_async_copy(genre_out.at[pl.ds(0, quarter)],
                                  rows_g, sem).wait()
            pltpu.sync_copy(rows_i, item_out.at[pl.ds(base + off, quarter)])
            pltpu.sync_copy(rows_k, key_out.at[pl.ds(base + off, quarter)])
            pltpu.sync_copy(rows_g, genre_out.at[pl.ds(base + off, quarter)])

        issue(0)
        issue(1)
        flush(0)
        issue(2)
        flush(1)
        issue(3)
        flush(2)
        flush(3)

    return gatherpack


def _gelu(x):
    return 0.5 * x * (1.0 + lax.erf(x * 0.7071067811865476))


def _dot(a, b):
    return jnp.dot(a, b, precision=lax.Precision.HIGHEST,
                   preferred_element_type=jnp.float32)


def _mlp_body(items, keys, genres, audio, wa, ba, w1i, w1k, w1g, w1a,
              b1, w2, b2, out):
    a = _gelu(_dot(audio[...], wa[...]) + ba[...])
    h = _dot(items[...], w1i[...])
    h = h + _dot(keys[...], w1k[...])
    h = h + _dot(genres[...], w1g[...])
    h = h + _dot(a, w1a[...])
    h = _gelu(h + b1[...])
    out[...] = _dot(h, w2[...]) + b2[...]


def kernel(item_ids, key_ids, genre_ids, audio_cont, item_emb, key_emb,
           genre_emb, W_audio, b_audio, W1, b1, W2, b2):
    n = item_ids.shape[0]
    d_item = item_emb.shape[1]
    d_small = key_emb.shape[1]
    d_pack = 128
    d_audio = W_audio.shape[1]
    d_hid = W1.shape[1]
    d_out = W2.shape[1]

    gatherpack = _build_gather(n, d_item, d_small)
    items, keys, genres = gatherpack(
        item_ids.astype(jnp.int32), key_ids.astype(jnp.int32),
        genre_ids.astype(jnp.int32), item_emb, key_emb, genre_emb)

    w1i = W1[:d_item]
    w1k = W1[d_item:d_item + d_small]
    w1g = W1[d_item + d_small:d_item + 2 * d_small]
    w1a = W1[d_item + 2 * d_small:]

    bn = min(n, 2048)
    grid = (n // bn,)

    def row_spec(d):
        return pl.BlockSpec((bn, d), lambda i: (i, 0))

    def rep_spec(r, c):
        return pl.BlockSpec((r, c), lambda i: (0, 0))

    return pl.pallas_call(
        _mlp_body,
        grid=grid,
        in_specs=[
            row_spec(d_item), row_spec(d_small), row_spec(d_small),
            row_spec(audio_cont.shape[1]),
            rep_spec(W_audio.shape[0], d_audio), rep_spec(1, d_audio),
            rep_spec(d_item, d_hid), rep_spec(d_small, d_hid),
            rep_spec(d_small, d_hid), rep_spec(d_audio, d_hid),
            rep_spec(1, d_hid),
            rep_spec(d_hid, d_out), rep_spec(1, d_out),
        ],
        out_specs=row_spec(d_out),
        out_shape=jax.ShapeDtypeStruct((n, d_out), jnp.float32),
    )(items, keys, genres, audio_cont,
      W_audio, b_audio.reshape(1, -1),
      w1i, w1k, w1g, w1a, b1.reshape(1, -1),
      W2, b2.reshape(1, -1))


# speedup vs baseline: 2.4762x; 1.2213x over previous
import functools

import jax
import jax.numpy as jnp
from jax import lax
from jax.experimental import pallas as pl
from jax.experimental.pallas import tpu as pltpu
from jax.experimental.pallas import tpu_sc as plsc

_NC = 2
_NS = 16
_NW = _NC * _NS


@functools.lru_cache(maxsize=None)
def _build_gather(n, d_item, d_small):
    bpw = n // _NW
    mesh = plsc.VectorSubcoreMesh(core_axis_name="c", subcore_axis_name="s")

    @functools.partial(
        pl.kernel,
        mesh=mesh,
        compiler_params=pltpu.CompilerParams(use_tc_tiling_on_sc=True),
        out_type=(
            jax.ShapeDtypeStruct((n, d_item), jnp.float32),
            jax.ShapeDtypeStruct((n, d_small), jnp.float32),
        ),
        scratch_types=[
            pltpu.VMEM((bpw,), jnp.int32),
            pltpu.VMEM((bpw,), jnp.int32),
            pltpu.VMEM((bpw // 4, d_item), jnp.float32),
            pltpu.VMEM((bpw // 4, d_small), jnp.float32),
            pltpu.VMEM((bpw // 4, d_item), jnp.float32),
            pltpu.VMEM((bpw // 4, d_small), jnp.float32),
            pltpu.SemaphoreType.DMA,
            pltpu.SemaphoreType.DMA,
        ],
    )
    def gather2(item_ids, genre_ids, item_emb, genre_emb,
                item_out, genre_out,
                sid_i, sid_g, ri0, rg0, ri1, rg1, sem0, sem1):
        wid = lax.axis_index("s") * _NC + lax.axis_index("c")
        base = wid * bpw
        quarter = bpw // 4
        pltpu.sync_copy(item_ids.at[pl.ds(base, bpw)], sid_i)
        pltpu.sync_copy(genre_ids.at[pl.ds(base, bpw)], sid_g)

        bufsets = ((ri0, rg0, sem0), (ri1, rg1, sem1))

        def issue(p):
            rows_i, rows_g, sem = bufsets[p % 2]
            lo = p * quarter

            def body(g, carry):
                iv = sid_i[pl.ds(lo + g * 16, 16)]
                gv = sid_g[pl.ds(lo + g * 16, 16)]
                for j in range(16):
                    r = g * 16 + j
                    pltpu.async_copy(item_emb.at[pl.ds(iv[j], 1), :],
                                     rows_i.at[pl.ds(r, 1), :], sem)
                    pltpu.async_copy(genre_emb.at[pl.ds(gv[j], 1), :],
                                     rows_g.at[pl.ds(r, 1), :], sem)
                return carry

            lax.fori_loop(0, quarter // 16, body, 0)

        def flush(p):
            rows_i, rows_g, sem = bufsets[p % 2]
            off = p * quarter
            pltpu.make_async_copy(item_out.at[pl.ds(0, quarter)],
                                  rows_i, sem).wait()
            pltpu.make_async_copy(genre_out.at[pl.ds(0, quarter)],
                                  rows_g, sem).wait()
            pltpu.sync_copy(rows_i, item_out.at[pl.ds(base + off, quarter)])
            pltpu.sync_copy(rows_g, genre_out.at[pl.ds(base + off, quarter)])

        issue(0)
        issue(1)
        flush(0)
        issue(2)
        flush(1)
        issue(3)
        flush(2)
        flush(3)

    return gather2


def _gelu(x):
    return 0.5 * x * (1.0 + lax.erf(x * 0.7071067811865476))


_HI = lax.Precision.DEFAULT


def _dot_t(at, w):
    return lax.dot_general(at, w, (((0,), (0,)), ((), ())),
                           precision=_HI, preferred_element_type=jnp.float32)


def _mlp_body(items, genres, kid, audio_t, kt, wa, ba,
              w1i, w1k, w1g, w1a, b1, w2t, b2, out, *, n_keys):
    a = _gelu(_dot_t(audio_t[...], wa[...]) + ba[...])
    onehot = (kid[...] == lax.broadcasted_iota(
        jnp.int32, (1, n_keys), 1)).astype(jnp.float32)
    keys = lax.dot_general(onehot, kt[...], (((1,), (1,)), ((), ())),
                           precision=_HI, preferred_element_type=jnp.float32)
    h = jnp.dot(items[...], w1i[...], precision=_HI,
                preferred_element_type=jnp.float32)
    h = h + jnp.dot(keys, w1k[...], precision=_HI,
                    preferred_element_type=jnp.float32)
    h = h + jnp.dot(genres[...], w1g[...], precision=_HI,
                    preferred_element_type=jnp.float32)
    h = h + jnp.dot(a, w1a[...], precision=_HI,
                    preferred_element_type=jnp.float32)
    h = _gelu(h + b1[...])
    out[...] = lax.dot_general(h, w2t[...], (((1,), (1,)), ((), ())),
                               precision=_HI,
                               preferred_element_type=jnp.float32) + b2[...]


def kernel(item_ids, key_ids, genre_ids, audio_cont, item_emb, key_emb,
           genre_emb, W_audio, b_audio, W1, b1, W2, b2):
    n = item_ids.shape[0]
    d_item = item_emb.shape[1]
    d_small = key_emb.shape[1]
    n_keys = key_emb.shape[0]
    d_audio = W_audio.shape[1]
    d_hid = W1.shape[1]
    d_out = W2.shape[1]

    gather2 = _build_gather(n, d_item, d_small)
    items, genres = gather2(item_ids.astype(jnp.int32),
                            genre_ids.astype(jnp.int32), item_emb, genre_emb)

    w1i = W1[:d_item]
    w1k = W1[d_item:d_item + d_small]
    w1g = W1[d_item + d_small:d_item + 2 * d_small]
    w1a = W1[d_item + 2 * d_small:]

    bn = min(n, 2048)
    grid = (n // bn,)

    def row_spec(d):
        return pl.BlockSpec((bn, d), lambda i: (i, 0))

    def col_spec(d):
        return pl.BlockSpec((d, bn), lambda i: (0, i))

    def rep_spec(r, c):
        return pl.BlockSpec((r, c), lambda i: (0, 0))

    return pl.pallas_call(
        functools.partial(_mlp_body, n_keys=n_keys),
        grid=grid,
        in_specs=[
            row_spec(d_item), row_spec(d_small), row_spec(1),
            col_spec(audio_cont.shape[1]),
            rep_spec(d_small, n_keys),
            rep_spec(W_audio.shape[0], d_audio), rep_spec(1, d_audio),
            rep_spec(d_item, d_hid), rep_spec(d_small, d_hid),
            rep_spec(d_small, d_hid), rep_spec(d_audio, d_hid),
            rep_spec(1, d_hid),
            rep_spec(d_out, d_hid), rep_spec(1, d_out),
        ],
        out_specs=row_spec(d_out),
        out_shape=jax.ShapeDtypeStruct((n, d_out), jnp.float32),
    )(items, genres, key_ids.astype(jnp.int32).reshape(n, 1),
      audio_cont.T, key_emb.T,
      W_audio, b_audio.reshape(1, -1),
      w1i, w1k, w1g, w1a, b1.reshape(1, -1),
      W2.T, b2.reshape(1, -1))
